# baseline (speedup 1.0000x reference)
import functools

import jax
import jax.numpy as jnp
from jax import lax
from jax.experimental import pallas as pl
from jax.experimental.pallas import tpu as pltpu
from jax.experimental.pallas import tpu_sc as plsc

NBINS = (128, 64, 32)
SUM_H = 224
SUM_W = 221
N_POINTS = 131072
N_GOI = 4096
N_GENES = 100000

NC, NS, L = 2, 16, 16
NW = NC * NS

LEVELS = ((0, 0, 128), (128, 127, 64), (192, 190, 32))

ROWS_A = N_GOI // NW
PTS_W = N_POINTS // NW
CHUNK = 64
NCHUNK = PTS_W // CHUNK
GRP = CHUNK // L
UNROLL = 4
SKEW = CHUNK + 1

_SC_PARAMS = pltpu.CompilerParams(use_tc_tiling_on_sc=False,
                                  needs_layout_passes=False)


def _worker_id():
    return lax.axis_index("s") * NC + lax.axis_index("c")


def _gather_rows_body(hw_hbm, ww_hbm, goi_hbm, uh_out, uw_out,
                      idx_v, uh_v, uw_v, sem1, sem2):
    base = _worker_id() * ROWS_A
    pltpu.sync_copy(goi_hbm.at[pl.ds(base, ROWS_A)], idx_v)
    c1 = pltpu.async_copy(hw_hbm.at[idx_v], uh_v, sem1)
    c2 = pltpu.async_copy(ww_hbm.at[idx_v], uw_v, sem2)
    c1.wait()
    c2.wait()
    pltpu.sync_copy(uh_v, uh_out.at[pl.ds(base, ROWS_A)])
    pltpu.sync_copy(uw_v, uw_out.at[pl.ds(base, ROWS_A)])


_gather_rows = functools.partial(
    pl.kernel,
    compiler_params=_SC_PARAMS,
    out_type=(jax.ShapeDtypeStruct((N_GOI, SUM_H), jnp.float32),
              jax.ShapeDtypeStruct((N_GOI, SUM_W), jnp.float32)),
    mesh=plsc.VectorSubcoreMesh(core_axis_name="c", subcore_axis_name="s"),
    scratch_types=[
        pltpu.VMEM((ROWS_A,), jnp.int32),
        pltpu.VMEM((ROWS_A, SUM_H), jnp.float32),
        pltpu.VMEM((ROWS_A, SUM_W), jnp.float32),
        pltpu.SemaphoreType.DMA,
        pltpu.SemaphoreType.DMA,
    ],
)(_gather_rows_body)


def _softmax_body(uw_ref, w_ref):
    w_ref[...] = jnp.zeros_like(w_ref)
    for ho, wo, k in LEVELS:
        seg = uw_ref[:, wo:wo + k - 1]
        m = jnp.max(seg, axis=1, keepdims=True)
        e = jnp.exp(seg - m)
        w_ref[:, wo:wo + k - 1] = e / jnp.sum(e, axis=1, keepdims=True)


def _softmax_widths(uw):
    return pl.pallas_call(
        _softmax_body,
        out_shape=jax.ShapeDtypeStruct((N_GOI, SUM_H), jnp.float32),
    )(uw)


def _spline_body(x_hbm, lg_hbm, delta_hbm, uh_hbm, w_hbm, out_hbm, hp_hbm,
                 lg_all, x_all, uh_v, w_v, dh_v, e_t, w_t, out_v, hp_v, sems):
    wid = _worker_id()
    pbase = wid * PTS_W
    lane = lax.iota(jnp.int32, L)

    pltpu.sync_copy(lg_hbm.at[pl.ds(pbase, PTS_W)], lg_all)
    pltpu.sync_copy(x_hbm.at[pl.ds(pbase, PTS_W)], x_all)

    def issue(ci, b):
        off = pbase + ci * CHUNK
        idx = lg_all.at[pl.ds(ci * CHUNK, CHUNK)]
        pltpu.async_copy(uh_hbm.at[idx],
                         uh_v.at[pl.ds(b * CHUNK, CHUNK)], sems.at[0, b])
        pltpu.async_copy(w_hbm.at[idx],
                         w_v.at[pl.ds(b * CHUNK, CHUNK)], sems.at[1, b])
        pltpu.async_copy(delta_hbm.at[pl.ds(off, CHUNK)],
                         dh_v.at[pl.ds(b * CHUNK, CHUNK)], sems.at[2, b])

    def wait_in(ci, b):
        idx = lg_all.at[pl.ds(ci * CHUNK, CHUNK)]
        pltpu.make_async_copy(uh_hbm.at[idx],
                              uh_v.at[pl.ds(b * CHUNK, CHUNK)], sems.at[0, b]).wait()
        pltpu.make_async_copy(w_hbm.at[idx],
                              w_v.at[pl.ds(b * CHUNK, CHUNK)], sems.at[1, b]).wait()
        pltpu.make_async_copy(delta_hbm.at[pl.ds(0, CHUNK)],
                              dh_v.at[pl.ds(b * CHUNK, CHUNK)], sems.at[2, b]).wait()

    def wait_out(ci, b):
        off = pbase + ci * CHUNK
        pltpu.make_async_copy(out_v.at[b], out_hbm.at[pl.ds(off, CHUNK)],
                              sems.at[3, b]).wait()
        pltpu.make_async_copy(hp_v.at[b], hp_hbm.at[pl.ds(off, CHUNK)],
                              sems.at[4, b]).wait()

    def repack(b):
        lane_sk = lane * SKEW

        @plsc.parallel_loop(0, CHUNK, unroll=2)
        def _(r):
            rv = jnp.broadcast_to(r + b * CHUNK, (L,)).astype(jnp.int32)
            for c in range(SUM_H // L):
                colv = lane + c * L
                idx = lane_sk + jnp.broadcast_to(c * L * SKEW + r, (L,)).astype(jnp.int32)
                a = plsc.load_gather(uh_v, [rv, colv])
                d = plsc.load_gather(dh_v, [rv, colv])
                plsc.store_scatter(e_t, [idx], jnp.exp(a + d))
                wv = plsc.load_gather(w_v, [rv, colv])
                plsc.store_scatter(w_t, [idx], wv)

    def compute(ci, b):
        for g in range(GRP):
            rowg = lane + g * L
            xg = plsc.load_gather(
                x_all,
                [jnp.broadcast_to(ci * CHUNK + g * L, (L,)).astype(jnp.int32) + lane])
            hp = jnp.ones((L,), jnp.float32)
            for ho, wo, k in LEVELS:
                e0 = plsc.load_gather(
                    e_t, [rowg + jnp.broadcast_to(jnp.int32(ho * SKEW), (L,))])
                zeros = jnp.zeros((L,), jnp.float32)
                carry0 = (e0, zeros, zeros, zeros, jnp.ones((L,), jnp.float32),
                          e0, e0, zeros)

                def step(j, carry, ho=ho, wo=wo, rowg=rowg, xg=xg):
                    e_prev, cw, area, cl, cwd, ceL, ceR, cp = carry
                    ie = rowg + jnp.broadcast_to((ho + 1 + j) * SKEW, (L,)).astype(jnp.int32)
                    iw = rowg + jnp.broadcast_to((wo + j) * SKEW, (L,)).astype(jnp.int32)
                    e_next = plsc.load_gather(e_t, [ie])
                    wk = plsc.load_gather(w_t, [iw])
                    cond = cw <= xg
                    cl = jnp.where(cond, cw, cl)
                    cwd = jnp.where(cond, wk, cwd)
                    ceL = jnp.where(cond, e_prev, ceL)
                    ceR = jnp.where(cond, e_next, ceR)
                    cp = jnp.where(cond, area, cp)
                    area = area + (e_prev + e_next) * wk
                    cw = cw + wk
                    return (e_next, cw, area, cl, cwd, ceL, ceR, cp)

                nfull = (k - 1) // UNROLL * UNROLL
                carry = plsc.parallel_loop(
                    0, nfull, unroll=UNROLL, carry=carry0)(step)
                for j in range(nfull, k - 1):
                    carry = step(jnp.int32(j), carry)
                _, _, area, cl, cwd, ceL, ceR, cp = carry
                inv_area = 2.0 / area
                alpha = (xg - cl) / cwd
                d_e = ceR - ceL
                qa = 0.5 * d_e * inv_area * cwd
                qb = ceL * inv_area * cwd
                qc = 0.5 * cp * inv_area
                out = qa * alpha * alpha + qb * alpha + qc
                hp = hp * ((alpha * d_e + ceL) * inv_area)
                xg = jnp.clip(out, 0.0, 1.0)
            out_v[b, pl.ds(g * L, L)] = xg
            hp_v[b, pl.ds(g * L, L)] = hp
        off = pbase + ci * CHUNK
        pltpu.async_copy(out_v.at[b], out_hbm.at[pl.ds(off, CHUNK)], sems.at[3, b])
        pltpu.async_copy(hp_v.at[b], hp_hbm.at[pl.ds(off, CHUNK)], sems.at[4, b])

    issue(0, 0)
    issue(1, 1)

    def outer(co, _):
        for b in range(2):
            ci = co * 2 + b
            wait_in(ci, b)
            repack(b)

            @pl.when(ci + 2 < NCHUNK)
            def _():
                issue(ci + 2, b)

            @pl.when(ci >= 2)
            def _():
                wait_out(ci - 2, b)

            compute(ci, b)
        return 0

    lax.fori_loop(0, NCHUNK // 2, outer, 0)
    wait_out(NCHUNK - 2, 0)
    wait_out(NCHUNK - 1, 1)


_spline = functools.partial(
    pl.kernel,
    compiler_params=_SC_PARAMS,
    out_type=(jax.ShapeDtypeStruct((N_POINTS,), jnp.float32),
              jax.ShapeDtypeStruct((N_POINTS,), jnp.float32)),
    mesh=plsc.VectorSubcoreMesh(core_axis_name="c", subcore_axis_name="s"),
    scratch_types=[
        pltpu.VMEM((PTS_W,), jnp.int32),
        pltpu.VMEM((PTS_W,), jnp.float32),
        pltpu.VMEM((2 * CHUNK, SUM_H), jnp.float32),
        pltpu.VMEM((2 * CHUNK, SUM_H), jnp.float32),
        pltpu.VMEM((2 * CHUNK, SUM_H), jnp.float32),
        pltpu.VMEM((SUM_H * SKEW,), jnp.float32),
        pltpu.VMEM((SUM_H * SKEW,), jnp.float32),
        pltpu.VMEM((2, CHUNK), jnp.float32),
        pltpu.VMEM((2, CHUNK), jnp.float32),
        pltpu.SemaphoreType.DMA((5, 2)),
    ],
)(_spline_body)


def _log_body(hp_ref, out_ref):
    out_ref[...] = jnp.log(hp_ref[...])


def _log_tc(hp):
    r = pl.pallas_call(
        _log_body,
        out_shape=jax.ShapeDtypeStruct((512, 256), jnp.float32),
    )(hp.reshape(512, 256))
    return r.reshape(N_POINTS)


def kernel(x, genes_oi, local_gene_ix, delta, heights_weight, widths_weight):
    uh, uw = _gather_rows(heights_weight, widths_weight, genes_oi)
    w = _softmax_widths(uw)
    outputs, hprod = _spline(x, local_gene_ix, delta, uh, w)
    return outputs, _log_tc(hprod)


# speedup vs baseline: 1.0010x; 1.0010x over previous
import functools

import jax
import jax.numpy as jnp
from jax import lax
from jax.experimental import pallas as pl
from jax.experimental.pallas import tpu as pltpu
from jax.experimental.pallas import tpu_sc as plsc

NBINS = (128, 64, 32)
SUM_H = 224
SUM_W = 221
N_POINTS = 131072
N_GOI = 4096
N_GENES = 100000

NC, NS, L = 2, 16, 16
NW = NC * NS

LEVELS = ((0, 0, 128), (128, 127, 64), (192, 190, 32))

ROWS_A = N_GOI // NW
PTS_W = N_POINTS // NW
CHUNK = 64
NCHUNK = PTS_W // CHUNK
GRP = CHUNK // L
UNROLL = 4
SKEW = CHUNK + 1

_SC_PARAMS = pltpu.CompilerParams(use_tc_tiling_on_sc=False,
                                  needs_layout_passes=False)


def _worker_id():
    return lax.axis_index("s") * NC + lax.axis_index("c")


def _gather_rows_body(hw_hbm, ww_hbm, goi_hbm, uh_out, uw_out,
                      idx_v, uh_v, uw_v, sem1, sem2):
    base = _worker_id() * ROWS_A
    pltpu.sync_copy(goi_hbm.at[pl.ds(base, ROWS_A)], idx_v)
    c1 = pltpu.async_copy(hw_hbm.at[idx_v], uh_v, sem1)
    c2 = pltpu.async_copy(ww_hbm.at[idx_v], uw_v, sem2)
    c1.wait()
    c2.wait()
    pltpu.sync_copy(uh_v, uh_out.at[pl.ds(base, ROWS_A)])
    pltpu.sync_copy(uw_v, uw_out.at[pl.ds(base, ROWS_A)])


_gather_rows = functools.partial(
    pl.kernel,
    compiler_params=_SC_PARAMS,
    out_type=(jax.ShapeDtypeStruct((N_GOI, SUM_H), jnp.float32),
              jax.ShapeDtypeStruct((N_GOI, SUM_H), jnp.float32)),
    mesh=plsc.VectorSubcoreMesh(core_axis_name="c", subcore_axis_name="s"),
    scratch_types=[
        pltpu.VMEM((ROWS_A,), jnp.int32),
        pltpu.VMEM((ROWS_A, SUM_H), jnp.float32),
        pltpu.VMEM((ROWS_A, SUM_H), jnp.float32),
        pltpu.SemaphoreType.DMA,
        pltpu.SemaphoreType.DMA,
    ],
)(_gather_rows_body)


def _softmax_body(uw_ref, w_ref):
    w_ref[...] = jnp.zeros_like(w_ref)
    for ho, wo, k in LEVELS:
        seg = uw_ref[:, wo:wo + k - 1]
        m = jnp.max(seg, axis=1, keepdims=True)
        e = jnp.exp(seg - m)
        w_ref[:, wo:wo + k - 1] = e / jnp.sum(e, axis=1, keepdims=True)


def _softmax_widths(uw):
    return pl.pallas_call(
        _softmax_body,
        out_shape=jax.ShapeDtypeStruct((N_GOI, SUM_H), jnp.float32),
    )(uw)


def _spline_body(x_hbm, lg_hbm, delta_hbm, uh_hbm, w_hbm, out_hbm, hp_hbm,
                 lg_all, x_all, uh_v, w_v, dh_v, e_t, w_t, out_v, hp_v, sems):
    wid = _worker_id()
    pbase = wid * PTS_W
    lane = lax.iota(jnp.int32, L)

    pltpu.sync_copy(lg_hbm.at[pl.ds(pbase, PTS_W)], lg_all)
    pltpu.sync_copy(x_hbm.at[pl.ds(pbase, PTS_W)], x_all)

    def issue(ci, b):
        off = pbase + ci * CHUNK
        idx = lg_all.at[pl.ds(ci * CHUNK, CHUNK)]
        pltpu.async_copy(uh_hbm.at[idx],
                         uh_v.at[pl.ds(b * CHUNK, CHUNK)], sems.at[0, b])
        pltpu.async_copy(w_hbm.at[idx],
                         w_v.at[pl.ds(b * CHUNK, CHUNK)], sems.at[1, b])
        pltpu.async_copy(delta_hbm.at[pl.ds(off, CHUNK)],
                         dh_v.at[pl.ds(b * CHUNK, CHUNK)], sems.at[2, b])

    def wait_in(ci, b):
        idx = lg_all.at[pl.ds(ci * CHUNK, CHUNK)]
        pltpu.make_async_copy(uh_hbm.at[idx],
                              uh_v.at[pl.ds(b * CHUNK, CHUNK)], sems.at[0, b]).wait()
        pltpu.make_async_copy(w_hbm.at[idx],
                              w_v.at[pl.ds(b * CHUNK, CHUNK)], sems.at[1, b]).wait()
        pltpu.make_async_copy(delta_hbm.at[pl.ds(0, CHUNK)],
                              dh_v.at[pl.ds(b * CHUNK, CHUNK)], sems.at[2, b]).wait()

    def wait_out(ci, b):
        off = pbase + ci * CHUNK
        pltpu.make_async_copy(out_v.at[b], out_hbm.at[pl.ds(off, CHUNK)],
                              sems.at[3, b]).wait()
        pltpu.make_async_copy(hp_v.at[b], hp_hbm.at[pl.ds(off, CHUNK)],
                              sems.at[4, b]).wait()

    def repack(b):
        lane_sk = lane * SKEW

        @plsc.parallel_loop(0, CHUNK, unroll=2)
        def _(r):
            rv = jnp.broadcast_to(r + b * CHUNK, (L,)).astype(jnp.int32)
            for c in range(SUM_H // L):
                colv = lane + c * L
                idx = lane_sk + jnp.broadcast_to(c * L * SKEW + r, (L,)).astype(jnp.int32)
                a = plsc.load_gather(uh_v, [rv, colv])
                d = plsc.load_gather(dh_v, [rv, colv])
                plsc.store_scatter(e_t, [idx], jnp.exp(a + d))
                wv = plsc.load_gather(w_v, [rv, colv])
                plsc.store_scatter(w_t, [idx], wv)

    def compute(ci, b):
        for g in range(GRP):
            rowg = lane + g * L
            xg = plsc.load_gather(
                x_all,
                [jnp.broadcast_to(ci * CHUNK + g * L, (L,)).astype(jnp.int32) + lane])
            hp = jnp.ones((L,), jnp.float32)
            for ho, wo, k in LEVELS:
                e0 = plsc.load_gather(
                    e_t, [rowg + jnp.broadcast_to(jnp.int32(ho * SKEW), (L,))])
                zeros = jnp.zeros((L,), jnp.float32)
                carry0 = (e0, zeros, zeros, zeros, jnp.ones((L,), jnp.float32),
                          e0, e0, zeros)

                def step(j, carry, ho=ho, wo=wo, rowg=rowg, xg=xg):
                    e_prev, cw, area, cl, cwd, ceL, ceR, cp = carry
                    ie = rowg + jnp.broadcast_to((ho + 1 + j) * SKEW, (L,)).astype(jnp.int32)
                    iw = rowg + jnp.broadcast_to((wo + j) * SKEW, (L,)).astype(jnp.int32)
                    e_next = plsc.load_gather(e_t, [ie])
                    wk = plsc.load_gather(w_t, [iw])
                    cond = cw <= xg
                    cl = jnp.where(cond, cw, cl)
                    cwd = jnp.where(cond, wk, cwd)
                    ceL = jnp.where(cond, e_prev, ceL)
                    ceR = jnp.where(cond, e_next, ceR)
                    cp = jnp.where(cond, area, cp)
                    area = area + (e_prev + e_next) * wk
                    cw = cw + wk
                    return (e_next, cw, area, cl, cwd, ceL, ceR, cp)

                nfull = (k - 1) // UNROLL * UNROLL
                carry = plsc.parallel_loop(
                    0, nfull, unroll=UNROLL, carry=carry0)(step)
                for j in range(nfull, k - 1):
                    carry = step(jnp.int32(j), carry)
                _, _, area, cl, cwd, ceL, ceR, cp = carry
                inv_area = 2.0 / area
                alpha = (xg - cl) / cwd
                d_e = ceR - ceL
                qa = 0.5 * d_e * inv_area * cwd
                qb = ceL * inv_area * cwd
                qc = 0.5 * cp * inv_area
                out = qa * alpha * alpha + qb * alpha + qc
                hp = hp * ((alpha * d_e + ceL) * inv_area)
                xg = jnp.clip(out, 0.0, 1.0)
            out_v[b, pl.ds(g * L, L)] = xg
            hp_v[b, pl.ds(g * L, L)] = hp
        off = pbase + ci * CHUNK
        pltpu.async_copy(out_v.at[b], out_hbm.at[pl.ds(off, CHUNK)], sems.at[3, b])
        pltpu.async_copy(hp_v.at[b], hp_hbm.at[pl.ds(off, CHUNK)], sems.at[4, b])

    issue(0, 0)
    issue(1, 1)

    def outer(co, _):
        for b in range(2):
            ci = co * 2 + b
            wait_in(ci, b)
            repack(b)

            @pl.when(ci + 2 < NCHUNK)
            def _():
                issue(ci + 2, b)

            @pl.when(ci >= 2)
            def _():
                wait_out(ci - 2, b)

            compute(ci, b)
        return 0

    lax.fori_loop(0, NCHUNK // 2, outer, 0)
    wait_out(NCHUNK - 2, 0)
    wait_out(NCHUNK - 1, 1)


_spline = functools.partial(
    pl.kernel,
    compiler_params=_SC_PARAMS,
    out_type=(jax.ShapeDtypeStruct((N_POINTS,), jnp.float32),
              jax.ShapeDtypeStruct((N_POINTS,), jnp.float32)),
    mesh=plsc.VectorSubcoreMesh(core_axis_name="c", subcore_axis_name="s"),
    scratch_types=[
        pltpu.VMEM((PTS_W,), jnp.int32),
        pltpu.VMEM((PTS_W,), jnp.float32),
        pltpu.VMEM((2 * CHUNK, SUM_H), jnp.float32),
        pltpu.VMEM((2 * CHUNK, SUM_H), jnp.float32),
        pltpu.VMEM((2 * CHUNK, SUM_H), jnp.float32),
        pltpu.VMEM((SUM_H * SKEW,), jnp.float32),
        pltpu.VMEM((SUM_H * SKEW,), jnp.float32),
        pltpu.VMEM((2, CHUNK), jnp.float32),
        pltpu.VMEM((2, CHUNK), jnp.float32),
        pltpu.SemaphoreType.DMA((5, 2)),
    ],
)(_spline_body)


def _log_body(hp_ref, out_ref):
    out_ref[...] = jnp.log(hp_ref[...])


def _log_tc(hp):
    r = pl.pallas_call(
        _log_body,
        out_shape=jax.ShapeDtypeStruct((512, 256), jnp.float32),
    )(hp.reshape(512, 256))
    return r.reshape(N_POINTS)


def kernel(x, genes_oi, local_gene_ix, delta, heights_weight, widths_weight):
    wwp = jnp.pad(widths_weight, ((0, 0), (0, SUM_H - SUM_W)))
    uh, uw = _gather_rows(heights_weight, wwp, genes_oi)
    w = _softmax_widths(uw)
    outputs, hprod = _spline(x, local_gene_ix, delta, uh, w)
    return outputs, _log_tc(hprod)
